# baseline (speedup 1.0000x reference)
import jax
import jax.numpy as jnp
from jax.experimental import pallas as pl
from jax.experimental.pallas import tpu as pltpu

B, H, N, D = 32, 8, 32768, 64
K = 32
EPS = 1e-08
BN = 2048


def _sim_body(keys_ref, beta_ref, mem_ref, sim_ref):
    keys = keys_ref[0]
    mem = mem_ref[0]
    dot = jax.lax.dot_general(keys, mem, (((1,), (1,)), ((), ())))
    k_norm = jnp.clip(jnp.sqrt(jnp.sum(keys * keys, axis=-1, keepdims=True)), EPS, None)
    m_norm = jnp.clip(jnp.sqrt(jnp.sum(mem * mem, axis=-1)), EPS, None)
    sim = dot / (k_norm * m_norm[None, :] + EPS) * beta_ref[0, 0][:, None]
    sim_ref[0] = sim


def _sim_pallas(memory, read_keys, beta, interpret=False):
    return pl.pallas_call(
        _sim_body,
        grid=(B, N // BN),
        in_specs=[
            pl.BlockSpec((1, H, D), lambda b, n: (b, 0, 0)),
            pl.BlockSpec((1, 1, H), lambda b, n: (b, 0, 0)),
            pl.BlockSpec((1, BN, D), lambda b, n: (b, n, 0)),
        ],
        out_specs=pl.BlockSpec((1, H, BN), lambda b, n: (b, 0, n)),
        out_shape=jax.ShapeDtypeStruct((B, H, N), jnp.float32),
        interpret=interpret,
    )(read_keys, beta[:, None, :], memory)


def _merge_body(read_ref, W_ref, b_ref, out_ref):
    out_ref[...] = (
        jax.lax.dot_general(read_ref[...], W_ref[...], (((1,), (1,)), ((), ())))
        + b_ref[...]
    )


def _merge_pallas(read_flat, W, b, interpret=False):
    return pl.pallas_call(
        _merge_body,
        out_shape=jax.ShapeDtypeStruct((B, D), jnp.float32),
        interpret=interpret,
    )(read_flat, W, b[None, :])


def kernel(memory, read_keys, beta, W, b):
    sim = _sim_pallas(memory, read_keys, beta)
    sim_flat = sim.reshape(B * H, N)
    vals, idx = jax.lax.top_k(sim_flat, K)
    m = vals.max(axis=-1, keepdims=True)
    e = jnp.exp(vals - m)
    w = e / e.sum(axis=-1, keepdims=True)
    weights = jnp.zeros((B * H, N), jnp.float32)
    weights = weights.at[jnp.arange(B * H)[:, None], idx].set(w)
    weights = weights.reshape(B, H, N)
    rows = memory.reshape(B * N, D)[(jnp.arange(B * H)[:, None] // H) * N + idx]
    read = jnp.einsum('rk,rkd->rd', w, rows)
    read_flat = read.reshape(B, H * D)
    read_combined = _merge_pallas(read_flat, W, b)
    return (read_combined, weights)
